# baseline (speedup 1.0000x reference)
import functools

import jax
import jax.numpy as jnp
from jax import lax
from jax.experimental import pallas as pl
from jax.experimental.pallas import tpu as pltpu
from jax.experimental.pallas import tpu_sc as plsc


@functools.lru_cache(maxsize=None)
def _make_gather(V, D, B):
    info = plsc.get_sparse_core_info()
    NC, NS = info.num_cores, info.num_subcores
    NW = NC * NS
    assert B % NW == 0
    b_per_w = B // NW
    CH = 128
    assert b_per_w % CH == 0
    n_chunks = b_per_w // CH

    mesh = plsc.VectorSubcoreMesh(core_axis_name="c", subcore_axis_name="s")

    @functools.partial(
        pl.kernel,
        mesh=mesh,
        out_type=(
            jax.ShapeDtypeStruct((B, D), jnp.float32),
            jax.ShapeDtypeStruct((B, D), jnp.float32),
        ),
        scratch_types=[
            pltpu.VMEM((n_chunks, CH), jnp.int32),
            pltpu.VMEM((b_per_w, D), jnp.float32),
            pltpu.VMEM((b_per_w, D), jnp.float32),
            pltpu.SemaphoreType.DMA,
            pltpu.SemaphoreType.DMA,
        ],
        compiler_params=pltpu.CompilerParams(use_tc_tiling_on_sc=False),
    )
    def gathered(k_hbm, loc_hbm, scale_hbm, out_loc, out_scale,
                 idx_v, loc_v, scale_v, sem_l, sem_s):
        wid = lax.axis_index("s") * NC + lax.axis_index("c")
        base = wid * b_per_w
        pltpu.sync_copy(k_hbm.at[wid], idx_v)
        copies = []
        for j in range(n_chunks):
            copies.append(pltpu.async_copy(
                loc_hbm.at[idx_v.at[j]], loc_v.at[pl.ds(j * CH, CH)], sem_l))
            copies.append(pltpu.async_copy(
                scale_hbm.at[idx_v.at[j]], scale_v.at[pl.ds(j * CH, CH)],
                sem_s))
        for c in copies:
            c.wait()
        pltpu.sync_copy(loc_v, out_loc.at[pl.ds(base, b_per_w)])
        pltpu.sync_copy(scale_v, out_scale.at[pl.ds(base, b_per_w)])

    return gathered, NW, n_chunks, CH


def kernel(u, k, loc, scale):
    B, D = u.shape
    V = loc.shape[0]
    fn, NW, n_chunks, CH = _make_gather(V, D, B)
    k_r = k.astype(jnp.int32).reshape(NW, n_chunks, CH)
    loc_k, scale_k = fn(k_r, loc, scale)
    return loc_k, scale_k
